# baseline (speedup 1.0000x reference)
import functools

import jax
import jax.numpy as jnp
from jax.experimental import pallas as pl
from jax.experimental.pallas import tpu as pltpu


def _moe_kernel(xT_ref, gw_ref, bias_ref, wgu_ref, wd_ref, wsgu_ref, wsd_ref,
                out_ref, xb_s, comb_s, *, n_experts, ff):
    e = pl.program_id(1)
    BT = xT_ref.shape[1]

    @pl.when(e == 0)
    def _routing():
        x = xT_ref[...]
        xb_s[...] = x.astype(jnp.bfloat16)
        logits = jax.lax.dot_general(
            gw_ref[...], xb_s[...],
            (((1,), (0,)), ((), ())),
            preferred_element_type=jnp.float32)
        scores = jax.nn.sigmoid(logits)
        sr = scores + bias_ref[...]
        eidx = jax.lax.broadcasted_iota(jnp.int32, (n_experts, BT), 0)
        swapped = jnp.concatenate(
            [sr[1:2], sr[0:1], sr[3:4], sr[2:3],
             sr[5:6], sr[4:5], sr[7:8], sr[6:7]], axis=0)
        gsum = sr + swapped
        gidx = eidx // 2
        big = jnp.int32(99)
        m1 = jnp.max(gsum, axis=0, keepdims=True)
        g1 = jnp.min(jnp.where(gsum == m1, gidx, big), axis=0, keepdims=True)
        gsum2 = jnp.where(gidx == g1, -jnp.inf, gsum)
        m2 = jnp.max(gsum2, axis=0, keepdims=True)
        g2 = jnp.min(jnp.where(gsum2 == m2, gidx, big), axis=0, keepdims=True)
        allowed = (gidx == g1) | (gidx == g2)
        masked = jnp.where(allowed, sr, -jnp.inf)
        m1e = jnp.max(masked, axis=0, keepdims=True)
        e1 = jnp.min(jnp.where(masked == m1e, eidx, big), axis=0, keepdims=True)
        masked2 = jnp.where(eidx == e1, -jnp.inf, masked)
        m2e = jnp.max(masked2, axis=0, keepdims=True)
        e2 = jnp.min(jnp.where(masked2 == m2e, eidx, big), axis=0, keepdims=True)
        zero = jnp.float32(0.0)
        w1 = jnp.sum(jnp.where(eidx == e1, scores, zero), axis=0, keepdims=True)
        w2 = jnp.sum(jnp.where(eidx == e2, scores, zero), axis=0, keepdims=True)
        denom = w1 + w2 + jnp.float32(1e-20)
        comb_s[...] = (jnp.where(eidx == e1, w1, zero)
                       + jnp.where(eidx == e2, w2, zero)) / denom

    xb = xb_s[...]
    gu = jax.lax.dot_general(
        wgu_ref[0], xb, (((1,), (0,)), ((), ())),
        preferred_element_type=jnp.float32)
    g = gu[:ff]
    u = gu[ff:]
    act = (jax.nn.silu(g) * u).astype(jnp.bfloat16)
    oe = jax.lax.dot_general(
        wd_ref[0], act, (((1,), (0,)), ((), ())),
        preferred_element_type=jnp.float32)
    contrib = comb_s[pl.ds(e, 1), :] * oe

    @pl.when(e == 0)
    def _init():
        out_ref[...] = contrib

    @pl.when(e > 0)
    def _acc():
        out_ref[...] += contrib

    @pl.when(e == n_experts - 1)
    def _shared():
        sgu = jax.lax.dot_general(
            wsgu_ref[...], xb, (((1,), (0,)), ((), ())),
            preferred_element_type=jnp.float32)
        sg = sgu[:ff]
        su = sgu[ff:]
        sact = (jax.nn.silu(sg) * su).astype(jnp.bfloat16)
        sout = jax.lax.dot_general(
            wsd_ref[...], sact, (((1,), (0,)), ((), ())),
            preferred_element_type=jnp.float32)
        out_ref[...] += sout


def kernel(hidden_states, image_mask, audio_mask, gate_w, expert_bias,
           w_gate_up, w_down, w_shared_gate_up, w_shared_down):
    del image_mask, audio_mask
    B, S, H = hidden_states.shape
    T = B * S
    E = gate_w.shape[0]
    FF = w_down.shape[2]
    BT = 1024
    n_tb = T // BT

    xT = hidden_states.reshape(T, H).T
    bias = expert_bias.reshape(E, 1)
    gw_b = gate_w.astype(jnp.bfloat16)
    wgu_b = w_gate_up.astype(jnp.bfloat16)
    wd_b = w_down.astype(jnp.bfloat16)
    wsgu_b = w_shared_gate_up.astype(jnp.bfloat16)
    wsd_b = w_shared_down.astype(jnp.bfloat16)

    grid = (n_tb, E)
    outT = pl.pallas_call(
        functools.partial(_moe_kernel, n_experts=E, ff=FF),
        grid=grid,
        in_specs=[
            pl.BlockSpec((H, BT), lambda tb, e: (0, tb)),
            pl.BlockSpec((E, H), lambda tb, e: (0, 0)),
            pl.BlockSpec((E, 1), lambda tb, e: (0, 0)),
            pl.BlockSpec((1, 2 * FF, H), lambda tb, e: (e, 0, 0)),
            pl.BlockSpec((1, H, FF), lambda tb, e: (e, 0, 0)),
            pl.BlockSpec((2 * FF, H), lambda tb, e: (0, 0)),
            pl.BlockSpec((H, FF), lambda tb, e: (0, 0)),
        ],
        out_specs=pl.BlockSpec((H, BT), lambda tb, e: (0, tb)),
        out_shape=jax.ShapeDtypeStruct((H, T), jnp.float32),
        scratch_shapes=[
            pltpu.VMEM((H, BT), jnp.bfloat16),
            pltpu.VMEM((E, BT), jnp.float32),
        ],
        compiler_params=pltpu.CompilerParams(
            dimension_semantics=("parallel", "arbitrary"),
            vmem_limit_bytes=64 * 1024 * 1024),
    )(xT, gw_b, bias, wgu_b, wd_b, wsgu_b, wsd_b)

    return outT.T.reshape(B, S, H)


# speedup vs baseline: 1.2957x; 1.2957x over previous
import functools

import jax
import jax.numpy as jnp
from jax.experimental import pallas as pl
from jax.experimental.pallas import tpu as pltpu


def _moe_kernel(xb_ref, gw_ref, bias_ref, wgu_ref, wd_ref, wsgu_ref, wsd_ref,
                out_ref, comb_s, *, n_experts, ff):
    e = pl.program_id(0)
    BT = xb_ref.shape[1]

    @pl.when(e == 0)
    def _routing():
        xb = xb_ref[...]
        logits = jax.lax.dot_general(
            gw_ref[...], xb, (((1,), (0,)), ((), ())),
            preferred_element_type=jnp.float32)
        scores = jax.nn.sigmoid(logits)
        sr = scores + bias_ref[...]
        eidx = jax.lax.broadcasted_iota(jnp.int32, (n_experts, BT), 0)
        swapped = jnp.concatenate(
            [sr[1:2], sr[0:1], sr[3:4], sr[2:3],
             sr[5:6], sr[4:5], sr[7:8], sr[6:7]], axis=0)
        gsum = sr + swapped
        gidx = eidx // 2
        big = jnp.int32(99)
        m1 = jnp.max(gsum, axis=0, keepdims=True)
        g1 = jnp.min(jnp.where(gsum == m1, gidx, big), axis=0, keepdims=True)
        gsum2 = jnp.where(gidx == g1, -jnp.inf, gsum)
        m2 = jnp.max(gsum2, axis=0, keepdims=True)
        g2 = jnp.min(jnp.where(gsum2 == m2, gidx, big), axis=0, keepdims=True)
        allowed = (gidx == g1) | (gidx == g2)
        masked = jnp.where(allowed, sr, -jnp.inf)
        m1e = jnp.max(masked, axis=0, keepdims=True)
        e1 = jnp.min(jnp.where(masked == m1e, eidx, big), axis=0, keepdims=True)
        masked2 = jnp.where(eidx == e1, -jnp.inf, masked)
        m2e = jnp.max(masked2, axis=0, keepdims=True)
        e2 = jnp.min(jnp.where(masked2 == m2e, eidx, big), axis=0, keepdims=True)
        zero = jnp.float32(0.0)
        w1 = jnp.sum(jnp.where(eidx == e1, scores, zero), axis=0, keepdims=True)
        w2 = jnp.sum(jnp.where(eidx == e2, scores, zero), axis=0, keepdims=True)
        denom = w1 + w2 + jnp.float32(1e-20)
        comb_s[...] = (jnp.where(eidx == e1, w1, zero)
                       + jnp.where(eidx == e2, w2, zero)) / denom
        out_ref[...] = jnp.zeros_like(out_ref)

    wgu_b = wgu_ref[0].astype(jnp.bfloat16)
    wd_b = wd_ref[0].astype(jnp.bfloat16)
    n_chunks = 2
    ch = BT // n_chunks
    for c in range(n_chunks):
        sl = pl.ds(c * ch, ch)
        xc = xb_ref[:, c * ch:(c + 1) * ch]
        gu = jax.lax.dot_general(
            wgu_b, xc, (((1,), (0,)), ((), ())),
            preferred_element_type=jnp.float32)
        cw = comb_s[pl.ds(e, 1), c * ch:(c + 1) * ch]
        act = (jax.nn.silu(gu[:ff]) * gu[ff:] * cw).astype(jnp.bfloat16)
        oe = jax.lax.dot_general(
            wd_b, act, (((1,), (0,)), ((), ())),
            preferred_element_type=jnp.float32)
        out_ref[:, sl] += oe

    @pl.when(e == n_experts - 1)
    def _shared():
        wsgu_b = wsgu_ref[...].astype(jnp.bfloat16)
        wsd_b = wsd_ref[...].astype(jnp.bfloat16)
        for c in range(n_chunks):
            sl = pl.ds(c * ch, ch)
            xc = xb_ref[:, c * ch:(c + 1) * ch]
            sgu = jax.lax.dot_general(
                wsgu_b, xc, (((1,), (0,)), ((), ())),
                preferred_element_type=jnp.float32)
            sact = (jax.nn.silu(sgu[:ff]) * sgu[ff:]).astype(jnp.bfloat16)
            sout = jax.lax.dot_general(
                wsd_b, sact, (((1,), (0,)), ((), ())),
                preferred_element_type=jnp.float32)
            out_ref[:, sl] += sout


def kernel(hidden_states, image_mask, audio_mask, gate_w, expert_bias,
           w_gate_up, w_down, w_shared_gate_up, w_shared_down):
    del image_mask, audio_mask
    B, S, H = hidden_states.shape
    T = B * S
    E = gate_w.shape[0]
    FF = w_down.shape[2]

    xT = hidden_states.reshape(T, H).T.astype(jnp.bfloat16)
    bias = expert_bias.reshape(E, 1)
    gw_b = gate_w.astype(jnp.bfloat16)

    outT = pl.pallas_call(
        functools.partial(_moe_kernel, n_experts=E, ff=FF),
        grid=(E,),
        in_specs=[
            pl.BlockSpec((H, T), lambda e: (0, 0)),
            pl.BlockSpec((E, H), lambda e: (0, 0)),
            pl.BlockSpec((E, 1), lambda e: (0, 0)),
            pl.BlockSpec((1, 2 * FF, H), lambda e: (e, 0, 0)),
            pl.BlockSpec((1, H, FF), lambda e: (e, 0, 0)),
            pl.BlockSpec((2 * FF, H), lambda e: (0, 0)),
            pl.BlockSpec((H, FF), lambda e: (0, 0)),
        ],
        out_specs=pl.BlockSpec((H, T), lambda e: (0, 0)),
        out_shape=jax.ShapeDtypeStruct((H, T), jnp.float32),
        scratch_shapes=[
            pltpu.VMEM((E, T), jnp.float32),
        ],
        compiler_params=pltpu.CompilerParams(
            dimension_semantics=("arbitrary",),
            vmem_limit_bytes=64 * 1024 * 1024),
    )(xT, gw_b, bias, w_gate_up, w_down, w_shared_gate_up, w_shared_down)

    return outT.T.reshape(B, S, H)


# speedup vs baseline: 1.3041x; 1.0065x over previous
import functools

import jax
import jax.numpy as jnp
from jax.experimental import pallas as pl
from jax.experimental.pallas import tpu as pltpu


def _moe_kernel(xin_ref, gw_ref, bias_ref, wgu_ref, wd_ref, wsgu_ref, wsd_ref,
                out_ref, xb_ref, comb_s, *, n_experts, ff):
    e = pl.program_id(0)
    BT = xin_ref.shape[0]

    @pl.when(e == 0)
    def _routing():
        xb_ref[...] = xin_ref[...].T
        xb = xb_ref[...]
        logits = jax.lax.dot_general(
            gw_ref[...], xb, (((1,), (0,)), ((), ())),
            preferred_element_type=jnp.float32)
        scores = jax.nn.sigmoid(logits)
        sr = scores + bias_ref[...]
        eidx = jax.lax.broadcasted_iota(jnp.int32, (n_experts, BT), 0)
        swapped = jnp.concatenate(
            [sr[1:2], sr[0:1], sr[3:4], sr[2:3],
             sr[5:6], sr[4:5], sr[7:8], sr[6:7]], axis=0)
        gsum = sr + swapped
        gidx = eidx // 2
        big = jnp.int32(99)
        m1 = jnp.max(gsum, axis=0, keepdims=True)
        g1 = jnp.min(jnp.where(gsum == m1, gidx, big), axis=0, keepdims=True)
        gsum2 = jnp.where(gidx == g1, -jnp.inf, gsum)
        m2 = jnp.max(gsum2, axis=0, keepdims=True)
        g2 = jnp.min(jnp.where(gsum2 == m2, gidx, big), axis=0, keepdims=True)
        allowed = (gidx == g1) | (gidx == g2)
        masked = jnp.where(allowed, sr, -jnp.inf)
        m1e = jnp.max(masked, axis=0, keepdims=True)
        e1 = jnp.min(jnp.where(masked == m1e, eidx, big), axis=0, keepdims=True)
        masked2 = jnp.where(eidx == e1, -jnp.inf, masked)
        m2e = jnp.max(masked2, axis=0, keepdims=True)
        e2 = jnp.min(jnp.where(masked2 == m2e, eidx, big), axis=0, keepdims=True)
        zero = jnp.float32(0.0)
        w1 = jnp.sum(jnp.where(eidx == e1, scores, zero), axis=0, keepdims=True)
        w2 = jnp.sum(jnp.where(eidx == e2, scores, zero), axis=0, keepdims=True)
        denom = w1 + w2 + jnp.float32(1e-20)
        comb_s[...] = (jnp.where(eidx == e1, w1, zero)
                       + jnp.where(eidx == e2, w2, zero)) / denom
        out_ref[...] = jnp.zeros_like(out_ref)

    wgu_b = wgu_ref[0].astype(jnp.bfloat16)
    wd_b = wd_ref[0].astype(jnp.bfloat16)
    n_chunks = 2
    ch = BT // n_chunks
    for c in range(n_chunks):
        sl = pl.ds(c * ch, ch)
        xc = xb_ref[:, c * ch:(c + 1) * ch]
        gu = jax.lax.dot_general(
            wgu_b, xc, (((1,), (0,)), ((), ())),
            preferred_element_type=jnp.float32)
        cw = comb_s[pl.ds(e, 1), c * ch:(c + 1) * ch]
        act = (jax.nn.silu(gu[:ff]) * gu[ff:] * cw).astype(jnp.bfloat16)
        oe = jax.lax.dot_general(
            wd_b, act, (((1,), (0,)), ((), ())),
            preferred_element_type=jnp.float32)
        out_ref[:, sl] += oe

    @pl.when(e == n_experts - 1)
    def _shared():
        wsgu_b = wsgu_ref[...].astype(jnp.bfloat16)
        wsd_b = wsd_ref[...].astype(jnp.bfloat16)
        for c in range(n_chunks):
            sl = pl.ds(c * ch, ch)
            xc = xb_ref[:, c * ch:(c + 1) * ch]
            sgu = jax.lax.dot_general(
                wsgu_b, xc, (((1,), (0,)), ((), ())),
                preferred_element_type=jnp.float32)
            sact = (jax.nn.silu(sgu[:ff]) * sgu[ff:]).astype(jnp.bfloat16)
            sout = jax.lax.dot_general(
                wsd_b, sact, (((1,), (0,)), ((), ())),
                preferred_element_type=jnp.float32)
            out_ref[:, sl] += sout


def kernel(hidden_states, image_mask, audio_mask, gate_w, expert_bias,
           w_gate_up, w_down, w_shared_gate_up, w_shared_down):
    del image_mask, audio_mask
    B, S, H = hidden_states.shape
    T = B * S
    E = gate_w.shape[0]
    FF = w_down.shape[2]

    xb16 = hidden_states.reshape(T, H).astype(jnp.bfloat16)
    bias = expert_bias.reshape(E, 1)
    gw_b = gate_w.astype(jnp.bfloat16)

    outT = pl.pallas_call(
        functools.partial(_moe_kernel, n_experts=E, ff=FF),
        grid=(E,),
        in_specs=[
            pl.BlockSpec((T, H), lambda e: (0, 0)),
            pl.BlockSpec((E, H), lambda e: (0, 0)),
            pl.BlockSpec((E, 1), lambda e: (0, 0)),
            pl.BlockSpec((1, 2 * FF, H), lambda e: (e, 0, 0)),
            pl.BlockSpec((1, H, FF), lambda e: (e, 0, 0)),
            pl.BlockSpec((2 * FF, H), lambda e: (0, 0)),
            pl.BlockSpec((H, FF), lambda e: (0, 0)),
        ],
        out_specs=pl.BlockSpec((H, T), lambda e: (0, 0)),
        out_shape=jax.ShapeDtypeStruct((H, T), jnp.float32),
        scratch_shapes=[
            pltpu.VMEM((H, T), jnp.bfloat16),
            pltpu.VMEM((E, T), jnp.float32),
        ],
        compiler_params=pltpu.CompilerParams(
            dimension_semantics=("arbitrary",),
            vmem_limit_bytes=64 * 1024 * 1024),
    )(xb16, gw_b, bias, w_gate_up, w_down, w_shared_gate_up, w_shared_down)

    return outT.T.reshape(B, S, H)


# speedup vs baseline: 1.3596x; 1.0425x over previous
import functools

import jax
import jax.numpy as jnp
from jax.experimental import pallas as pl
from jax.experimental.pallas import tpu as pltpu


def _moe_kernel(xin_ref, gw_ref, bias_ref, wgu_ref, wd_ref, wsgu_ref, wsd_ref,
                out_ref, xb_ref, comb_s, *, n_experts, ff):
    e = pl.program_id(0)
    BT = xin_ref.shape[0]

    @pl.when(e == 0)
    def _routing():
        xb_ref[...] = xin_ref[...].T
        xb = xb_ref[...]
        logits = jax.lax.dot_general(
            gw_ref[...], xb, (((1,), (0,)), ((), ())),
            preferred_element_type=jnp.float32)
        scores = jax.nn.sigmoid(logits)
        sr = scores + bias_ref[...]
        eidx = jax.lax.broadcasted_iota(jnp.int32, (n_experts, BT), 0)
        swapped = jnp.concatenate(
            [sr[1:2], sr[0:1], sr[3:4], sr[2:3],
             sr[5:6], sr[4:5], sr[7:8], sr[6:7]], axis=0)
        gsum = sr + swapped
        gidx = eidx // 2
        big = jnp.int32(99)
        m1 = jnp.max(gsum, axis=0, keepdims=True)
        g1 = jnp.min(jnp.where(gsum == m1, gidx, big), axis=0, keepdims=True)
        gsum2 = jnp.where(gidx == g1, -jnp.inf, gsum)
        m2 = jnp.max(gsum2, axis=0, keepdims=True)
        g2 = jnp.min(jnp.where(gsum2 == m2, gidx, big), axis=0, keepdims=True)
        allowed = (gidx == g1) | (gidx == g2)
        masked = jnp.where(allowed, sr, -jnp.inf)
        m1e = jnp.max(masked, axis=0, keepdims=True)
        e1 = jnp.min(jnp.where(masked == m1e, eidx, big), axis=0, keepdims=True)
        masked2 = jnp.where(eidx == e1, -jnp.inf, masked)
        m2e = jnp.max(masked2, axis=0, keepdims=True)
        e2 = jnp.min(jnp.where(masked2 == m2e, eidx, big), axis=0, keepdims=True)
        zero = jnp.float32(0.0)
        w1 = jnp.sum(jnp.where(eidx == e1, scores, zero), axis=0, keepdims=True)
        w2 = jnp.sum(jnp.where(eidx == e2, scores, zero), axis=0, keepdims=True)
        denom = w1 + w2 + jnp.float32(1e-20)
        comb_s[...] = (jnp.where(eidx == e1, w1, zero)
                       + jnp.where(eidx == e2, w2, zero)) / denom
        out_ref[...] = jnp.zeros_like(out_ref)

    wgu_b = wgu_ref[0].astype(jnp.bfloat16)
    wd_b = wd_ref[0].astype(jnp.bfloat16)
    n_chunks = 2
    ch = BT // n_chunks
    for c in range(n_chunks):
        sl = pl.ds(c * ch, ch)
        xc = xb_ref[:, c * ch:(c + 1) * ch]
        g = jax.lax.dot_general(
            wgu_b[:ff], xc, (((1,), (0,)), ((), ())),
            preferred_element_type=jnp.float32)
        u = jax.lax.dot_general(
            wgu_b[ff:], xc, (((1,), (0,)), ((), ())),
            preferred_element_type=jnp.float32)
        cw = comb_s[pl.ds(e, 1), c * ch:(c + 1) * ch]
        act = (jax.nn.silu(g) * u * cw).astype(jnp.bfloat16)
        oe = jax.lax.dot_general(
            wd_b, act, (((1,), (0,)), ((), ())),
            preferred_element_type=jnp.float32)
        out_ref[:, sl] += oe

    @pl.when(e == n_experts - 1)
    def _shared():
        wsgu_b = wsgu_ref[...].astype(jnp.bfloat16)
        wsd_b = wsd_ref[...].astype(jnp.bfloat16)
        for c in range(n_chunks):
            sl = pl.ds(c * ch, ch)
            xc = xb_ref[:, c * ch:(c + 1) * ch]
            sgu = jax.lax.dot_general(
                wsgu_b, xc, (((1,), (0,)), ((), ())),
                preferred_element_type=jnp.float32)
            sact = (jax.nn.silu(sgu[:ff]) * sgu[ff:]).astype(jnp.bfloat16)
            sout = jax.lax.dot_general(
                wsd_b, sact, (((1,), (0,)), ((), ())),
                preferred_element_type=jnp.float32)
            out_ref[:, sl] += sout


def kernel(hidden_states, image_mask, audio_mask, gate_w, expert_bias,
           w_gate_up, w_down, w_shared_gate_up, w_shared_down):
    del image_mask, audio_mask
    B, S, H = hidden_states.shape
    T = B * S
    E = gate_w.shape[0]
    FF = w_down.shape[2]

    xb16 = hidden_states.reshape(T, H).astype(jnp.bfloat16)
    bias = expert_bias.reshape(E, 1)
    gw_b = gate_w.astype(jnp.bfloat16)

    outT = pl.pallas_call(
        functools.partial(_moe_kernel, n_experts=E, ff=FF),
        grid=(E,),
        in_specs=[
            pl.BlockSpec((T, H), lambda e: (0, 0)),
            pl.BlockSpec((E, H), lambda e: (0, 0)),
            pl.BlockSpec((E, 1), lambda e: (0, 0)),
            pl.BlockSpec((1, 2 * FF, H), lambda e: (e, 0, 0)),
            pl.BlockSpec((1, H, FF), lambda e: (e, 0, 0)),
            pl.BlockSpec((2 * FF, H), lambda e: (0, 0)),
            pl.BlockSpec((H, FF), lambda e: (0, 0)),
        ],
        out_specs=pl.BlockSpec((H, T), lambda e: (0, 0)),
        out_shape=jax.ShapeDtypeStruct((H, T), jnp.float32),
        scratch_shapes=[
            pltpu.VMEM((H, T), jnp.bfloat16),
            pltpu.VMEM((E, T), jnp.float32),
        ],
        compiler_params=pltpu.CompilerParams(
            dimension_semantics=("arbitrary",),
            vmem_limit_bytes=64 * 1024 * 1024),
    )(xb16, gw_b, bias, w_gate_up, w_down, w_shared_gate_up, w_shared_down)

    return outT.T.reshape(B, S, H)


# speedup vs baseline: 1.3740x; 1.0106x over previous
import functools

import jax
import jax.numpy as jnp
from jax.experimental import pallas as pl
from jax.experimental.pallas import tpu as pltpu


def _moe_kernel(xin_ref, gw_ref, bias_ref, wgu_ref, wd_ref, wsgu_ref, wsd_ref,
                out_ref, xb_ref, comb_s, *, n_experts, ff):
    e = pl.program_id(0)
    BT = xin_ref.shape[0]

    @pl.when(e == 0)
    def _routing():
        xb_ref[...] = xin_ref[...].T
        xb = xb_ref[...]
        logits = jax.lax.dot_general(
            gw_ref[...], xb, (((1,), (0,)), ((), ())),
            preferred_element_type=jnp.float32)
        scores = jax.nn.sigmoid(logits)
        sr = scores + bias_ref[...]
        eidx = jax.lax.broadcasted_iota(jnp.int32, (n_experts, BT), 0)
        swapped = jnp.concatenate(
            [sr[1:2], sr[0:1], sr[3:4], sr[2:3],
             sr[5:6], sr[4:5], sr[7:8], sr[6:7]], axis=0)
        gsum = sr + swapped
        gidx = eidx // 2
        big = jnp.int32(99)
        m1 = jnp.max(gsum, axis=0, keepdims=True)
        g1 = jnp.min(jnp.where(gsum == m1, gidx, big), axis=0, keepdims=True)
        gsum2 = jnp.where(gidx == g1, -jnp.inf, gsum)
        m2 = jnp.max(gsum2, axis=0, keepdims=True)
        g2 = jnp.min(jnp.where(gsum2 == m2, gidx, big), axis=0, keepdims=True)
        allowed = (gidx == g1) | (gidx == g2)
        masked = jnp.where(allowed, sr, -jnp.inf)
        m1e = jnp.max(masked, axis=0, keepdims=True)
        e1 = jnp.min(jnp.where(masked == m1e, eidx, big), axis=0, keepdims=True)
        masked2 = jnp.where(eidx == e1, -jnp.inf, masked)
        m2e = jnp.max(masked2, axis=0, keepdims=True)
        e2 = jnp.min(jnp.where(masked2 == m2e, eidx, big), axis=0, keepdims=True)
        zero = jnp.float32(0.0)
        w1 = jnp.sum(jnp.where(eidx == e1, scores, zero), axis=0, keepdims=True)
        w2 = jnp.sum(jnp.where(eidx == e2, scores, zero), axis=0, keepdims=True)
        denom = w1 + w2 + jnp.float32(1e-20)
        comb_s[...] = (jnp.where(eidx == e1, w1, zero)
                       + jnp.where(eidx == e2, w2, zero)) / denom
        out_ref[...] = jnp.zeros_like(out_ref)

    wgu_b = wgu_ref[0].astype(jnp.bfloat16)
    wd_b = wd_ref[0].astype(jnp.bfloat16)
    n_chunks = 2
    ch = BT // n_chunks
    for c in range(n_chunks):
        sl = pl.ds(c * ch, ch)
        xc = xb_ref[:, c * ch:(c + 1) * ch]
        g = jax.lax.dot_general(
            wgu_b[:ff], xc, (((1,), (0,)), ((), ())),
            preferred_element_type=jnp.float32)
        u = jax.lax.dot_general(
            wgu_b[ff:], xc, (((1,), (0,)), ((), ())),
            preferred_element_type=jnp.float32)
        cw = comb_s[pl.ds(e, 1), c * ch:(c + 1) * ch]
        act = (jax.nn.silu(g) * u * cw).astype(jnp.bfloat16)
        oe = jax.lax.dot_general(
            wd_b, act, (((1,), (0,)), ((), ())),
            preferred_element_type=jnp.float32)
        out_ref[:, sl] += oe

    @pl.when(e == n_experts - 1)
    def _shared():
        wsgu_b = wsgu_ref[...].astype(jnp.bfloat16)
        wsd_b = wsd_ref[...].astype(jnp.bfloat16)
        for c in range(n_chunks):
            sl = pl.ds(c * ch, ch)
            xc = xb_ref[:, c * ch:(c + 1) * ch]
            sg = jax.lax.dot_general(
                wsgu_b[:ff], xc, (((1,), (0,)), ((), ())),
                preferred_element_type=jnp.float32)
            su = jax.lax.dot_general(
                wsgu_b[ff:], xc, (((1,), (0,)), ((), ())),
                preferred_element_type=jnp.float32)
            sact = (jax.nn.silu(sg) * su).astype(jnp.bfloat16)
            sout = jax.lax.dot_general(
                wsd_b, sact, (((1,), (0,)), ((), ())),
                preferred_element_type=jnp.float32)
            out_ref[:, sl] += sout


def kernel(hidden_states, image_mask, audio_mask, gate_w, expert_bias,
           w_gate_up, w_down, w_shared_gate_up, w_shared_down):
    del image_mask, audio_mask
    B, S, H = hidden_states.shape
    T = B * S
    E = gate_w.shape[0]
    FF = w_down.shape[2]

    xb16 = hidden_states.reshape(T, H).astype(jnp.bfloat16)
    bias = expert_bias.reshape(E, 1)
    gw_b = gate_w.astype(jnp.bfloat16)

    outT = pl.pallas_call(
        functools.partial(_moe_kernel, n_experts=E, ff=FF),
        grid=(E,),
        in_specs=[
            pl.BlockSpec((T, H), lambda e: (0, 0)),
            pl.BlockSpec((E, H), lambda e: (0, 0)),
            pl.BlockSpec((E, 1), lambda e: (0, 0)),
            pl.BlockSpec((1, 2 * FF, H), lambda e: (e, 0, 0)),
            pl.BlockSpec((1, H, FF), lambda e: (e, 0, 0)),
            pl.BlockSpec((2 * FF, H), lambda e: (0, 0)),
            pl.BlockSpec((H, FF), lambda e: (0, 0)),
        ],
        out_specs=pl.BlockSpec((H, T), lambda e: (0, 0)),
        out_shape=jax.ShapeDtypeStruct((H, T), jnp.float32),
        scratch_shapes=[
            pltpu.VMEM((H, T), jnp.bfloat16),
            pltpu.VMEM((E, T), jnp.float32),
        ],
        compiler_params=pltpu.CompilerParams(
            dimension_semantics=("arbitrary",),
            vmem_limit_bytes=64 * 1024 * 1024),
    )(xb16, gw_b, bias, w_gate_up, w_down, w_shared_gate_up, w_shared_down)

    return outT.T.reshape(B, S, H)
